# Initial kernel scaffold; baseline (speedup 1.0000x reference)
import jax
import jax.numpy as jnp
from jax.experimental import pallas as pl


def kernel(input, adj, W, a):
    raise NotImplementedError("write your pallas kernel here")


# speedup vs baseline: 39.9214x; 39.9214x over previous
import functools

import jax
import jax.numpy as jnp
from jax import lax
from jax.experimental import pallas as pl
from jax.experimental.pallas import tpu as pltpu
from jax.experimental.pallas import tpu_sc as plsc

N = 10000
E = 320000
IN_F = 128
HEADS = 8
OUT_PH = 16
D = HEADS * OUT_PH
HH = 4
NEG_SLOPE = 0.2

NC = 2
NS = 16
NW = NC * NS
EPW = E // NW
CH = 80
NCHUNK = EPW // CH
ZTILES = 10
ROWS_PT = N // ZTILES
ZCH = 200
TROWS = N // NS


def _prep_body(x_ref, m0_ref, m1_ref, t0_ref, t1_ref):
    x = x_ref[...]
    t0_ref[...] = jnp.dot(x, m0_ref[...], preferred_element_type=jnp.float32)
    t1_ref[...] = jnp.dot(x, m1_ref[...], preferred_element_type=jnp.float32)


def _prep(x, m0, m1):
    return pl.pallas_call(
        _prep_body,
        out_shape=[
            jax.ShapeDtypeStruct((N, IN_F), jnp.float32),
            jax.ShapeDtypeStruct((N, IN_F), jnp.float32),
        ],
    )(x, m0, m1)


_GATHER_DNUMS = lax.GatherDimensionNumbers(
    offset_dims=(), collapsed_slice_dims=(0,), start_index_map=(0,))


def _edge_body(src_hbm, dst_hbm, t0_hbm, t1_hbm, zu_hbm,
               u0_out, u1_out, sidx, didx, fs, hd, u_sp, sem):
    c = lax.axis_index("c")
    t = lax.axis_index("s")
    wid = c * NS + t
    base = t * ROWS_PT

    shift4 = (lax.iota(jnp.int32, 16) % HH) + HH

    def zero_u():
        @pl.when(t < ZTILES)
        def _z():
            for j in range(ROWS_PT // ZCH):
                sl = pl.ds(base + j * ZCH, ZCH)
                pltpu.sync_copy(zu_hbm.at[sl], u_sp.at[sl])

    def run_half(t_hbm, u_out):
        def chunk_body(i, carry):
            off = wid * EPW + i * CH
            pltpu.sync_copy(src_hbm.at[pl.ds(off, CH)], sidx)
            pltpu.sync_copy(dst_hbm.at[pl.ds(off, CH)], didx)
            cp1 = pltpu.make_async_copy(t_hbm.at[sidx], fs, sem)
            cp2 = pltpu.make_async_copy(t_hbm.at[didx], hd, sem)
            cp1.start()
            cp2.start()
            cp1.wait()
            cp2.wait()

            def edge_body(k, carry2):
                fblk = fs[k, pl.ds(HH * OUT_PH, 16)]
                gblk = hd[k, pl.ds(HH * OUT_PH, 16)]
                gsh = lax.gather(gblk, shift4[:, None], _GATHER_DNUMS, (1,),
                                 mode=lax.GatherScatterMode.PROMISE_IN_BOUNDS)
                e = fblk + gsh
                e = jnp.maximum(e, NEG_SLOPE * e)
                exr = jnp.exp(e)
                for h in range(HH):
                    sp = lax.gather(
                        exr, jnp.full((16, 1), h, jnp.int32), _GATHER_DNUMS,
                        (1,), mode=lax.GatherScatterMode.PROMISE_IN_BOUNDS)
                    sl = pl.ds(h * OUT_PH, OUT_PH)
                    sl2 = pl.ds((HH + h) * OUT_PH, OUT_PH)
                    hd[k, sl] = hd[k, sl] * sp
                    hd[k, sl2] = hd[k, sl2] * 0.0 + sp
                return carry2

            lax.fori_loop(0, CH, edge_body, 0, unroll=2)
            pltpu.sync_copy(hd, u_sp.at[sidx], add=True)
            return carry

        lax.fori_loop(0, NCHUNK, chunk_body, 0)
        plsc.subcore_barrier()

        @pl.when(t < ZTILES)
        def _dump():
            for j in range(ROWS_PT // ZCH):
                sl = pl.ds(base + j * ZCH, ZCH)
                pltpu.sync_copy(u_sp.at[sl], u_out.at[c, sl])

        plsc.subcore_barrier()

    zero_u()
    plsc.subcore_barrier()
    run_half(t0_hbm, u0_out)
    zero_u()
    plsc.subcore_barrier()
    run_half(t1_hbm, u1_out)


_edge_kernel = functools.partial(
    pl.kernel,
    out_type=[
        jax.ShapeDtypeStruct((NC, N, D), jnp.float32),
        jax.ShapeDtypeStruct((NC, N, D), jnp.float32),
    ],
    mesh=plsc.VectorSubcoreMesh(core_axis_name="c", subcore_axis_name="s"),
    scratch_types=[
        pltpu.VMEM((CH,), jnp.int32),
        pltpu.VMEM((CH,), jnp.int32),
        pltpu.VMEM((CH, D), jnp.float32),
        pltpu.VMEM((CH, D), jnp.float32),
        pltpu.VMEM_SHARED((N, D), jnp.float32),
        pltpu.SemaphoreType.DMA,
    ],
)(_edge_body)


def _norm_body(ua0_ref, ub0_ref, ua1_ref, ub1_ref, p64_ref, k0_ref, k1_ref,
               o_ref):
    u0 = ua0_ref[...] + ub0_ref[...]
    u1 = ua1_ref[...] + ub1_ref[...]
    p64 = p64_ref[...]
    r0 = 1.0 / (jnp.dot(u0, p64, preferred_element_type=jnp.float32) + 1e-16)
    r1 = 1.0 / (jnp.dot(u1, p64, preferred_element_type=jnp.float32) + 1e-16)
    o_ref[...] = (
        jnp.dot(u0 * r0, k0_ref[...], preferred_element_type=jnp.float32)
        + jnp.dot(u1 * r1, k1_ref[...], preferred_element_type=jnp.float32))


def _norm(u0, u1):
    half = HH * OUT_PH
    eye = jnp.eye(half, dtype=jnp.float32)
    zz = jnp.zeros((half, half), jnp.float32)
    p64 = jnp.block([[zz, zz], [eye, zz]])
    k0 = jnp.block([[eye, zz], [zz, zz]])
    k1 = jnp.block([[zz, eye], [zz, zz]])
    return pl.pallas_call(
        _norm_body,
        out_shape=jax.ShapeDtypeStruct((N, D), jnp.float32),
    )(u0[0], u0[1], u1[0], u1[1], p64, k0, k1)


def kernel(input, adj, W, a):
    x = input
    a1 = a[:, :OUT_PH]
    a2 = a[:, OUT_PH:]
    fvec = jnp.einsum("hij,hj->hi", W, a1)
    gvec = jnp.einsum("hij,hj->hi", W, a2)

    def build_m(c):
        wc = jnp.transpose(W[c * HH:(c + 1) * HH], (1, 0, 2)).reshape(IN_F, HH * OUT_PH)
        fc = jnp.transpose(fvec[c * HH:(c + 1) * HH])
        gc = jnp.transpose(gvec[c * HH:(c + 1) * HH])
        pad = jnp.zeros((IN_F, IN_F - HH * OUT_PH - 2 * HH), jnp.float32)
        return jnp.concatenate([wc, fc, gc, pad], axis=1)

    t0, t1 = _prep(x, build_m(0), build_m(1))
    src = adj[0]
    dst = adj[1]
    zu = jnp.zeros((N, D), jnp.float32)
    u0, u1 = _edge_kernel(src, dst, t0, t1, zu)
    return _norm(u0, u1)


# speedup vs baseline: 57.4379x; 1.4388x over previous
import functools

import jax
import jax.numpy as jnp
from jax import lax
from jax.experimental import pallas as pl
from jax.experimental.pallas import tpu as pltpu
from jax.experimental.pallas import tpu_sc as plsc

N = 10000
E = 320000
IN_F = 128
HEADS = 8
OUT_PH = 16
D = HEADS * OUT_PH
HH = 4
NEG_SLOPE = 0.2

NC = 2
NS = 16
NW = NC * NS
EPW = E // NW
CH = 80
NCHUNK = EPW // CH
ZTILES = 10
ROWS_PT = N // ZTILES
ZCH = 200
TROWS = N // NS


def _prep_body(x_ref, m0_ref, m1_ref, t0_ref, t1_ref):
    x = x_ref[...]
    t0_ref[...] = jnp.dot(x, m0_ref[...], preferred_element_type=jnp.float32)
    t1_ref[...] = jnp.dot(x, m1_ref[...], preferred_element_type=jnp.float32)


def _prep(x, m0, m1):
    return pl.pallas_call(
        _prep_body,
        out_shape=[
            jax.ShapeDtypeStruct((N, IN_F), jnp.float32),
            jax.ShapeDtypeStruct((N, IN_F), jnp.float32),
        ],
    )(x, m0, m1)


_GATHER_DNUMS = lax.GatherDimensionNumbers(
    offset_dims=(), collapsed_slice_dims=(0,), start_index_map=(0,))


def _edge_body(src_hbm, dst_hbm, t0_hbm, t1_hbm, zu_hbm,
               u0_out, u1_out, sidx0, didx0, fs0, hd0,
               sidx1, didx1, fs1, hd1, u_sp, sem0, sem1):
    c = lax.axis_index("c")
    t = lax.axis_index("s")
    wid = c * NS + t
    base = t * ROWS_PT

    shift4 = (lax.iota(jnp.int32, 16) % HH) + HH
    slots = ((sidx0, didx0, fs0, hd0, sem0), (sidx1, didx1, fs1, hd1, sem1))

    def zero_u():
        @pl.when(t < ZTILES)
        def _z():
            for j in range(ROWS_PT // ZCH):
                sl = pl.ds(base + j * ZCH, ZCH)
                pltpu.sync_copy(zu_hbm.at[sl], u_sp.at[sl])

    def run_half(t_hbm, u_out):
        ebase = wid * EPW

        def prefetch(i, slot):
            sidx, didx, fs, hd, sem = slots[slot]
            off = ebase + i * CH
            pltpu.sync_copy(src_hbm.at[pl.ds(off, CH)], sidx)
            pltpu.sync_copy(dst_hbm.at[pl.ds(off, CH)], didx)
            pltpu.make_async_copy(t_hbm.at[sidx], fs, sem).start()
            pltpu.make_async_copy(t_hbm.at[didx], hd, sem).start()

        def process(slot):
            sidx, didx, fs, hd, sem = slots[slot]
            pltpu.make_async_copy(t_hbm.at[sidx], fs, sem).wait()
            pltpu.make_async_copy(t_hbm.at[didx], hd, sem).wait()

            def edge_body(k, carry2):
                fblk = fs[k, pl.ds(HH * OUT_PH, 16)]
                gblk = hd[k, pl.ds(HH * OUT_PH, 16)]
                gsh = lax.gather(gblk, shift4[:, None], _GATHER_DNUMS, (1,),
                                 mode=lax.GatherScatterMode.PROMISE_IN_BOUNDS)
                e = fblk + gsh
                e = jnp.maximum(e, NEG_SLOPE * e)
                exr = jnp.exp(e)
                for h in range(HH):
                    sp = lax.gather(
                        exr, jnp.full((16, 1), h, jnp.int32), _GATHER_DNUMS,
                        (1,), mode=lax.GatherScatterMode.PROMISE_IN_BOUNDS)
                    sl = pl.ds(h * OUT_PH, OUT_PH)
                    sl2 = pl.ds((HH + h) * OUT_PH, OUT_PH)
                    hd[k, sl] = hd[k, sl] * sp
                    hd[k, sl2] = hd[k, sl2] * 0.0 + sp
                return carry2

            lax.fori_loop(0, CH, edge_body, 0, unroll=2)
            pltpu.sync_copy(hd, u_sp.at[sidx], add=True)

        prefetch(0, 0)

        def pair_body(j, carry):
            cidx = 2 * j
            prefetch(cidx + 1, 1)
            process(0)
            prefetch(cidx + 2, 0)
            process(1)
            return carry

        lax.fori_loop(0, (NCHUNK - 1) // 2, pair_body, 0)
        process(0)
        plsc.subcore_barrier()

        @pl.when(t < ZTILES)
        def _dump():
            for j in range(ROWS_PT // ZCH):
                sl = pl.ds(base + j * ZCH, ZCH)
                pltpu.sync_copy(u_sp.at[sl], u_out.at[c, sl])

        plsc.subcore_barrier()

    zero_u()
    plsc.subcore_barrier()
    run_half(t0_hbm, u0_out)
    zero_u()
    plsc.subcore_barrier()
    run_half(t1_hbm, u1_out)


_edge_kernel = functools.partial(
    pl.kernel,
    out_type=[
        jax.ShapeDtypeStruct((NC, N, D), jnp.float32),
        jax.ShapeDtypeStruct((NC, N, D), jnp.float32),
    ],
    mesh=plsc.VectorSubcoreMesh(core_axis_name="c", subcore_axis_name="s"),
    scratch_types=[
        pltpu.VMEM((CH,), jnp.int32),
        pltpu.VMEM((CH,), jnp.int32),
        pltpu.VMEM((CH, D), jnp.float32),
        pltpu.VMEM((CH, D), jnp.float32),
        pltpu.VMEM((CH,), jnp.int32),
        pltpu.VMEM((CH,), jnp.int32),
        pltpu.VMEM((CH, D), jnp.float32),
        pltpu.VMEM((CH, D), jnp.float32),
        pltpu.VMEM_SHARED((N, D), jnp.float32),
        pltpu.SemaphoreType.DMA,
        pltpu.SemaphoreType.DMA,
    ],
)(_edge_body)


def _norm_body(ua0_ref, ub0_ref, ua1_ref, ub1_ref, p64_ref, k0_ref, k1_ref,
               o_ref):
    u0 = ua0_ref[...] + ub0_ref[...]
    u1 = ua1_ref[...] + ub1_ref[...]
    p64 = p64_ref[...]
    r0 = 1.0 / (jnp.dot(u0, p64, preferred_element_type=jnp.float32) + 1e-16)
    r1 = 1.0 / (jnp.dot(u1, p64, preferred_element_type=jnp.float32) + 1e-16)
    o_ref[...] = (
        jnp.dot(u0 * r0, k0_ref[...], preferred_element_type=jnp.float32)
        + jnp.dot(u1 * r1, k1_ref[...], preferred_element_type=jnp.float32))


def _norm(u0, u1):
    half = HH * OUT_PH
    eye = jnp.eye(half, dtype=jnp.float32)
    zz = jnp.zeros((half, half), jnp.float32)
    p64 = jnp.block([[zz, zz], [eye, zz]])
    k0 = jnp.block([[eye, zz], [zz, zz]])
    k1 = jnp.block([[zz, eye], [zz, zz]])
    return pl.pallas_call(
        _norm_body,
        out_shape=jax.ShapeDtypeStruct((N, D), jnp.float32),
    )(u0[0], u0[1], u1[0], u1[1], p64, k0, k1)


def kernel(input, adj, W, a):
    x = input
    a1 = a[:, :OUT_PH]
    a2 = a[:, OUT_PH:]
    fvec = jnp.einsum("hij,hj->hi", W, a1)
    gvec = jnp.einsum("hij,hj->hi", W, a2)

    def build_m(c):
        wc = jnp.transpose(W[c * HH:(c + 1) * HH], (1, 0, 2)).reshape(IN_F, HH * OUT_PH)
        fc = jnp.transpose(fvec[c * HH:(c + 1) * HH])
        gc = jnp.transpose(gvec[c * HH:(c + 1) * HH])
        pad = jnp.zeros((IN_F, IN_F - HH * OUT_PH - 2 * HH), jnp.float32)
        return jnp.concatenate([wc, fc, gc, pad], axis=1)

    t0, t1 = _prep(x, build_m(0), build_m(1))
    src = adj[0]
    dst = adj[1]
    zu = jnp.zeros((N, D), jnp.float32)
    u0, u1 = _edge_kernel(src, dst, t0, t1, zu)
    return _norm(u0, u1)


# speedup vs baseline: 77.3233x; 1.3462x over previous
import functools

import jax
import jax.numpy as jnp
from jax import lax
from jax.experimental import pallas as pl
from jax.experimental.pallas import tpu as pltpu
from jax.experimental.pallas import tpu_sc as plsc

N = 10000
E = 320000
IN_F = 128
HEADS = 8
OUT_PH = 16
D = HEADS * OUT_PH
HH = 4
NEG_SLOPE = 0.2

NC = 2
NS = 16
NW = NC * NS
EPW = E // NW
CH = 80
NCHUNK = EPW // CH
ZTILES = 10
ROWS_PT = N // ZTILES
ZCH = 200
TROWS = N // NS


def _prep_body(x_ref, m0_ref, m1_ref, t0_ref, t1_ref):
    x = x_ref[...]
    t0_ref[...] = jnp.dot(x, m0_ref[...], preferred_element_type=jnp.float32)
    t1_ref[...] = jnp.dot(x, m1_ref[...], preferred_element_type=jnp.float32)


def _prep(x, m0, m1):
    return pl.pallas_call(
        _prep_body,
        out_shape=[
            jax.ShapeDtypeStruct((N, IN_F), jnp.float32),
            jax.ShapeDtypeStruct((N, IN_F), jnp.float32),
        ],
    )(x, m0, m1)


_GATHER_DNUMS = lax.GatherDimensionNumbers(
    offset_dims=(), collapsed_slice_dims=(0,), start_index_map=(0,))


def _edge_body(src_hbm, dst_hbm, t0_hbm, t1_hbm, zu_hbm,
               u0_out, u1_out,
               sidxq0, didxq0, sidxq1, didxq1,
               sidxq2, didxq2, sidxq3, didxq3,
               fs0, hd0, fs1, hd1, u_sp,
               isem0, isem1, isem2, isem3, sem0, sem1):
    c = lax.axis_index("c")
    t = lax.axis_index("s")
    wid = c * NS + t
    base = t * ROWS_PT

    shift4 = (lax.iota(jnp.int32, 16) % HH) + HH
    idxq = ((sidxq0, didxq0, isem0), (sidxq1, didxq1, isem1),
            (sidxq2, didxq2, isem2), (sidxq3, didxq3, isem3))
    gslots = ((fs0, hd0, sem0), (fs1, hd1, sem1))

    def zero_u():
        @pl.when(t < ZTILES)
        def _z():
            for j in range(ROWS_PT // ZCH):
                sl = pl.ds(base + j * ZCH, ZCH)
                pltpu.sync_copy(zu_hbm.at[sl], u_sp.at[sl])

    def run_half(t_hbm, u_out):
        ebase = wid * EPW

        def idx_start2(off_words, q):
            sidx, didx, isem = idxq[q]
            pltpu.make_async_copy(src_hbm.at[pl.ds(off_words, CH)], sidx,
                                  isem).start()
            pltpu.make_async_copy(dst_hbm.at[pl.ds(off_words, CH)], didx,
                                  isem).start()

        def gather_start2(off_words, q, g):
            sidx, didx, isem = idxq[q]
            pltpu.make_async_copy(src_hbm.at[pl.ds(off_words, CH)], sidx,
                                  isem).wait()
            pltpu.make_async_copy(dst_hbm.at[pl.ds(off_words, CH)], didx,
                                  isem).wait()
            fs, hd, sem = gslots[g]
            pltpu.make_async_copy(t_hbm.at[sidx], fs, sem).start()
            pltpu.make_async_copy(t_hbm.at[didx], hd, sem).start()

        def process2(q, g):
            sidx, didx, _ = idxq[q]
            fs, hd, sem = gslots[g]
            pltpu.make_async_copy(t_hbm.at[sidx], fs, sem).wait()
            pltpu.make_async_copy(t_hbm.at[didx], hd, sem).wait()

            def edge_body(k, carry2):
                fblk = fs[k, pl.ds(HH * OUT_PH, 16)]
                gblk = hd[k, pl.ds(HH * OUT_PH, 16)]
                gsh = lax.gather(gblk, shift4[:, None], _GATHER_DNUMS, (1,),
                                 mode=lax.GatherScatterMode.PROMISE_IN_BOUNDS)
                e = fblk + gsh
                e = jnp.maximum(e, NEG_SLOPE * e)
                exr = jnp.exp(e)
                for h in range(HH):
                    sp = lax.gather(
                        exr, jnp.full((16, 1), h, jnp.int32), _GATHER_DNUMS,
                        (1,), mode=lax.GatherScatterMode.PROMISE_IN_BOUNDS)
                    sl = pl.ds(h * OUT_PH, OUT_PH)
                    sl2 = pl.ds((HH + h) * OUT_PH, OUT_PH)
                    hd[k, sl] = hd[k, sl] * sp
                    hd[k, sl2] = hd[k, sl2] * 0.0 + sp
                return carry2

            lax.fori_loop(0, CH, edge_body, 0, unroll=2)
            pltpu.sync_copy(hd, u_sp.at[sidx], add=True)

        idx_start2(ebase, 0)
        idx_start2(ebase + CH, 1)
        gather_start2(ebase, 0, 0)

        def quad_body(j, carry):
            off = ebase + 4 * j * CH
            for k in range(4):
                idx_start2(off + (k + 2) * CH, (k + 2) % 4)
                gather_start2(off + (k + 1) * CH, (k + 1) % 4, (k + 1) % 2)
                process2(k % 4, k % 2)
            return carry

        lax.fori_loop(0, NCHUNK // 4 - 1, quad_body, 0)

        off = ebase + (NCHUNK - 5) * CH
        for k in range(5):
            cc = NCHUNK - 5 + k
            if cc + 2 < NCHUNK:
                idx_start2(off + (k + 2) * CH, (cc + 2) % 4)
            if cc + 1 < NCHUNK:
                gather_start2(off + (k + 1) * CH, (cc + 1) % 4, (cc + 1) % 2)
            process2(cc % 4, cc % 2)

        plsc.subcore_barrier()

        @pl.when(t < ZTILES)
        def _dump():
            for j in range(ROWS_PT // ZCH):
                sl = pl.ds(base + j * ZCH, ZCH)
                pltpu.sync_copy(u_sp.at[sl], u_out.at[c, sl])

        plsc.subcore_barrier()

    zero_u()
    plsc.subcore_barrier()
    run_half(t0_hbm, u0_out)
    zero_u()
    plsc.subcore_barrier()
    run_half(t1_hbm, u1_out)


_edge_kernel = functools.partial(
    pl.kernel,
    out_type=[
        jax.ShapeDtypeStruct((NC, N, D), jnp.float32),
        jax.ShapeDtypeStruct((NC, N, D), jnp.float32),
    ],
    mesh=plsc.VectorSubcoreMesh(core_axis_name="c", subcore_axis_name="s"),
    scratch_types=(
        [pltpu.VMEM((CH,), jnp.int32) for _ in range(8)]
        + [
            pltpu.VMEM((CH, D), jnp.float32),
            pltpu.VMEM((CH, D), jnp.float32),
            pltpu.VMEM((CH, D), jnp.float32),
            pltpu.VMEM((CH, D), jnp.float32),
            pltpu.VMEM_SHARED((N, D), jnp.float32),
        ]
        + [pltpu.SemaphoreType.DMA for _ in range(6)]
    ),
)(_edge_body)


def _norm_body(ua0_ref, ub0_ref, ua1_ref, ub1_ref, p64_ref, k0_ref, k1_ref,
               o_ref):
    u0 = ua0_ref[...] + ub0_ref[...]
    u1 = ua1_ref[...] + ub1_ref[...]
    p64 = p64_ref[...]
    r0 = 1.0 / (jnp.dot(u0, p64, preferred_element_type=jnp.float32) + 1e-16)
    r1 = 1.0 / (jnp.dot(u1, p64, preferred_element_type=jnp.float32) + 1e-16)
    o_ref[...] = (
        jnp.dot(u0 * r0, k0_ref[...], preferred_element_type=jnp.float32)
        + jnp.dot(u1 * r1, k1_ref[...], preferred_element_type=jnp.float32))


def _norm(u0, u1):
    half = HH * OUT_PH
    eye = jnp.eye(half, dtype=jnp.float32)
    zz = jnp.zeros((half, half), jnp.float32)
    p64 = jnp.block([[zz, zz], [eye, zz]])
    k0 = jnp.block([[eye, zz], [zz, zz]])
    k1 = jnp.block([[zz, eye], [zz, zz]])
    return pl.pallas_call(
        _norm_body,
        out_shape=jax.ShapeDtypeStruct((N, D), jnp.float32),
    )(u0[0], u0[1], u1[0], u1[1], p64, k0, k1)


def kernel(input, adj, W, a):
    x = input
    a1 = a[:, :OUT_PH]
    a2 = a[:, OUT_PH:]
    fvec = jnp.einsum("hij,hj->hi", W, a1)
    gvec = jnp.einsum("hij,hj->hi", W, a2)

    def build_m(c):
        wc = jnp.transpose(W[c * HH:(c + 1) * HH], (1, 0, 2)).reshape(IN_F, HH * OUT_PH)
        fc = jnp.transpose(fvec[c * HH:(c + 1) * HH])
        gc = jnp.transpose(gvec[c * HH:(c + 1) * HH])
        pad = jnp.zeros((IN_F, IN_F - HH * OUT_PH - 2 * HH), jnp.float32)
        return jnp.concatenate([wc, fc, gc, pad], axis=1)

    t0, t1 = _prep(x, build_m(0), build_m(1))
    src = adj[0]
    dst = adj[1]
    zu = jnp.zeros((N, D), jnp.float32)
    u0, u1 = _edge_kernel(src, dst, t0, t1, zu)
    return _norm(u0, u1)


# speedup vs baseline: 80.3466x; 1.0391x over previous
import functools

import jax
import jax.numpy as jnp
from jax import lax
from jax.experimental import pallas as pl
from jax.experimental.pallas import tpu as pltpu
from jax.experimental.pallas import tpu_sc as plsc

N = 10000
E = 320000
IN_F = 128
HEADS = 8
OUT_PH = 16
D = HEADS * OUT_PH
HH = 4
NEG_SLOPE = 0.2

NC = 2
NS = 16
NW = NC * NS
EPW = E // NW
CH = 40
NCHUNK = EPW // CH
ZTILES = 10
ROWS_PT = N // ZTILES
ZCH = 200
TROWS = N // NS


def _prep_body(x_ref, m0_ref, m1_ref, t0_ref, t1_ref):
    x = x_ref[...]
    t0_ref[...] = jnp.dot(x, m0_ref[...], preferred_element_type=jnp.float32)
    t1_ref[...] = jnp.dot(x, m1_ref[...], preferred_element_type=jnp.float32)


def _prep(x, m0, m1):
    return pl.pallas_call(
        _prep_body,
        out_shape=[
            jax.ShapeDtypeStruct((N, IN_F), jnp.float32),
            jax.ShapeDtypeStruct((N, IN_F), jnp.float32),
        ],
    )(x, m0, m1)


_GATHER_DNUMS = lax.GatherDimensionNumbers(
    offset_dims=(), collapsed_slice_dims=(0,), start_index_map=(0,))


def _edge_body(src_hbm, dst_hbm, t0_hbm, t1_hbm, zu_hbm,
               u0_out, u1_out,
               sidxq0, didxq0, sidxq1, didxq1, sidxq2, didxq2,
               sidxq3, didxq3, sidxq4, didxq4, sidxq5, didxq5,
               fs0, hd0, fs1, hd1, fs2, hd2, u_sp,
               isem0, isem1, isem2, isem3, isem4, isem5,
               sem0, sem1, sem2, ssem0, ssem1, ssem2):
    c = lax.axis_index("c")
    t = lax.axis_index("s")
    wid = c * NS + t
    base = t * ROWS_PT

    shift4 = (lax.iota(jnp.int32, 16) % HH) + HH
    idxq = ((sidxq0, didxq0, isem0), (sidxq1, didxq1, isem1),
            (sidxq2, didxq2, isem2), (sidxq3, didxq3, isem3),
            (sidxq4, didxq4, isem4), (sidxq5, didxq5, isem5))
    gslots = ((fs0, hd0, sem0, ssem0), (fs1, hd1, sem1, ssem1),
              (fs2, hd2, sem2, ssem2))

    def zero_u():
        @pl.when(t < ZTILES)
        def _z():
            for j in range(ROWS_PT // ZCH):
                sl = pl.ds(base + j * ZCH, ZCH)
                pltpu.sync_copy(zu_hbm.at[sl], u_sp.at[sl])

    def run_half(t_hbm, u_out):
        ebase = wid * EPW

        def idx_start2(off_words, q):
            sidx, didx, isem = idxq[q]
            pltpu.make_async_copy(src_hbm.at[pl.ds(off_words, CH)], sidx,
                                  isem).start()
            pltpu.make_async_copy(dst_hbm.at[pl.ds(off_words, CH)], didx,
                                  isem).start()

        def scatter_drain(m6, m3):
            _, hd, _, ssem = gslots[m3]
            sidx = idxq[m6][0]
            pltpu.make_async_copy(hd, u_sp.at[sidx], ssem).wait()

        def gather_start2(off_words, q, g, drain):
            if drain:
                scatter_drain((q + 3) % 6, g)
            sidx, didx, isem = idxq[q]
            pltpu.make_async_copy(src_hbm.at[pl.ds(off_words, CH)], sidx,
                                  isem).wait()
            pltpu.make_async_copy(dst_hbm.at[pl.ds(off_words, CH)], didx,
                                  isem).wait()
            fs, hd, sem, _ = gslots[g % 3]
            pltpu.make_async_copy(t_hbm.at[sidx], fs, sem).start()
            pltpu.make_async_copy(t_hbm.at[didx], hd, sem).start()

        def process2(q, g):
            sidx, didx, _ = idxq[q]
            fs, hd, sem, ssem = gslots[g]
            pltpu.make_async_copy(t_hbm.at[sidx], fs, sem).wait()
            pltpu.make_async_copy(t_hbm.at[didx], hd, sem).wait()

            def edge_body(k, carry2):
                fblk = fs[k, pl.ds(HH * OUT_PH, 16)]
                gblk = hd[k, pl.ds(HH * OUT_PH, 16)]
                gsh = lax.gather(gblk, shift4[:, None], _GATHER_DNUMS, (1,),
                                 mode=lax.GatherScatterMode.PROMISE_IN_BOUNDS)
                e = fblk + gsh
                e = jnp.maximum(e, NEG_SLOPE * e)
                exr = jnp.exp(e)
                for h in range(HH):
                    sp = lax.gather(
                        exr, jnp.full((16, 1), h, jnp.int32), _GATHER_DNUMS,
                        (1,), mode=lax.GatherScatterMode.PROMISE_IN_BOUNDS)
                    sl = pl.ds(h * OUT_PH, OUT_PH)
                    sl2 = pl.ds((HH + h) * OUT_PH, OUT_PH)
                    hd[k, sl] = hd[k, sl] * sp
                    hd[k, sl2] = hd[k, sl2] * 0.0 + sp
                return carry2

            lax.fori_loop(0, CH, edge_body, 0, unroll=2)
            pltpu.make_async_copy(hd, u_sp.at[sidx], ssem).start(add=True)

        idx_start2(ebase, 0)
        idx_start2(ebase + CH, 1)
        gather_start2(ebase, 0, 0, False)
        idx_start2(ebase + 2 * CH, 2)
        gather_start2(ebase + CH, 1, 1, False)
        process2(0, 0)
        idx_start2(ebase + 3 * CH, 3)
        gather_start2(ebase + 2 * CH, 2, 2, False)
        process2(1, 1)

        nhex = (NCHUNK - 4) // 6
        pstart = 2 + 6 * nhex

        def hex_body(j, carry):
            off = ebase + 6 * j * CH
            for k in range(6):
                cc = 2 + k
                idx_start2(off + (cc + 2) * CH, (cc + 2) % 6)
                gather_start2(off + (cc + 1) * CH, (cc + 1) % 6,
                              (cc + 1) % 3, True)
                process2(cc % 6, cc % 3)
            return carry

        lax.fori_loop(0, nhex, hex_body, 0)

        for cc in range(pstart, NCHUNK):
            if cc + 2 < NCHUNK:
                idx_start2(ebase + (cc + 2) * CH, (cc + 2) % 6)
            if cc + 1 < NCHUNK:
                gather_start2(ebase + (cc + 1) * CH, (cc + 1) % 6,
                              (cc + 1) % 3, True)
            process2(cc % 6, cc % 3)
        for gg in range(NCHUNK - 3, NCHUNK):
            scatter_drain(gg % 6, gg % 3)

        plsc.subcore_barrier()

        @pl.when(t < ZTILES)
        def _dump():
            for j in range(ROWS_PT // ZCH):
                sl = pl.ds(base + j * ZCH, ZCH)
                pltpu.sync_copy(u_sp.at[sl], u_out.at[c, sl])

        plsc.subcore_barrier()

    zero_u()
    plsc.subcore_barrier()
    run_half(t0_hbm, u0_out)
    zero_u()
    plsc.subcore_barrier()
    run_half(t1_hbm, u1_out)


_edge_kernel = functools.partial(
    pl.kernel,
    out_type=[
        jax.ShapeDtypeStruct((NC, N, D), jnp.float32),
        jax.ShapeDtypeStruct((NC, N, D), jnp.float32),
    ],
    mesh=plsc.VectorSubcoreMesh(core_axis_name="c", subcore_axis_name="s"),
    scratch_types=(
        [pltpu.VMEM((CH,), jnp.int32) for _ in range(12)]
        + [pltpu.VMEM((CH, D), jnp.float32) for _ in range(6)]
        + [pltpu.VMEM_SHARED((N, D), jnp.float32)]
        + [pltpu.SemaphoreType.DMA for _ in range(12)]
    ),
)(_edge_body)


def _norm_body(ua0_ref, ub0_ref, ua1_ref, ub1_ref, p64_ref, k0_ref, k1_ref,
               o_ref):
    u0 = ua0_ref[...] + ub0_ref[...]
    u1 = ua1_ref[...] + ub1_ref[...]
    p64 = p64_ref[...]
    r0 = 1.0 / (jnp.dot(u0, p64, preferred_element_type=jnp.float32) + 1e-16)
    r1 = 1.0 / (jnp.dot(u1, p64, preferred_element_type=jnp.float32) + 1e-16)
    o_ref[...] = (
        jnp.dot(u0 * r0, k0_ref[...], preferred_element_type=jnp.float32)
        + jnp.dot(u1 * r1, k1_ref[...], preferred_element_type=jnp.float32))


def _norm(u0, u1):
    half = HH * OUT_PH
    eye = jnp.eye(half, dtype=jnp.float32)
    zz = jnp.zeros((half, half), jnp.float32)
    p64 = jnp.block([[zz, zz], [eye, zz]])
    k0 = jnp.block([[eye, zz], [zz, zz]])
    k1 = jnp.block([[zz, eye], [zz, zz]])
    return pl.pallas_call(
        _norm_body,
        out_shape=jax.ShapeDtypeStruct((N, D), jnp.float32),
    )(u0[0], u0[1], u1[0], u1[1], p64, k0, k1)


def kernel(input, adj, W, a):
    x = input
    a1 = a[:, :OUT_PH]
    a2 = a[:, OUT_PH:]
    fvec = jnp.einsum("hij,hj->hi", W, a1)
    gvec = jnp.einsum("hij,hj->hi", W, a2)

    def build_m(c):
        wc = jnp.transpose(W[c * HH:(c + 1) * HH], (1, 0, 2)).reshape(IN_F, HH * OUT_PH)
        fc = jnp.transpose(fvec[c * HH:(c + 1) * HH])
        gc = jnp.transpose(gvec[c * HH:(c + 1) * HH])
        pad = jnp.zeros((IN_F, IN_F - HH * OUT_PH - 2 * HH), jnp.float32)
        return jnp.concatenate([wc, fc, gc, pad], axis=1)

    t0, t1 = _prep(x, build_m(0), build_m(1))
    src = adj[0]
    dst = adj[1]
    zu = jnp.zeros((N, D), jnp.float32)
    u0, u1 = _edge_kernel(src, dst, t0, t1, zu)
    return _norm(u0, u1)
